# Initial kernel scaffold; baseline (speedup 1.0000x reference)
import jax
import jax.numpy as jnp
from jax.experimental import pallas as pl


def kernel(ee_scores, er_scores, entity_labels, relation_types):
    raise NotImplementedError("write your pallas kernel here")


# speedup vs baseline: 101.7570x; 101.7570x over previous
import functools

import jax
import jax.numpy as jnp
from jax.experimental import pallas as pl
from jax.experimental.pallas import tpu as pltpu

N = 5000
R = 16
K = 20
BI = 200
THR = 0.5
NEG = float("-inf")


def _rowmax_kernel(ee_ref, p_ref, pt_ref, pts_ref, out_ref):
    blk = pl.program_id(0)
    s = jax.nn.sigmoid(ee_ref[...])
    rows = blk * BI + jax.lax.broadcasted_iota(jnp.int32, (BI, N), 0)
    cols = jax.lax.broadcasted_iota(jnp.int32, (BI, N), 1)
    use_a = cols < rows
    for r in range(R):
        y = jnp.where(use_a, s * pt_ref[r : r + 1, :], s * pts_ref[r : r + 1, :])
        m = jnp.max(y, axis=1, keepdims=True) * p_ref[:, r : r + 1]
        out_ref[:, r : r + 1] = jnp.where(m > THR, m, NEG)


def _pairsel_kernel(rm_ref, rows_ref, rels_ref):
    vals = rm_ref[...]
    lin = (
        jax.lax.broadcasted_iota(jnp.int32, (N, R), 0) * R
        + jax.lax.broadcasted_iota(jnp.int32, (N, R), 1)
    )
    opos = (
        jax.lax.broadcasted_iota(jnp.int32, (8, 128), 0) * 128
        + jax.lax.broadcasted_iota(jnp.int32, (8, 128), 1)
    )
    rows_acc = jnp.zeros((8, 128), jnp.int32)
    rels_acc = jnp.zeros((8, 128), jnp.int32)
    for k in range(K):
        m = jnp.max(vals)
        sel = jnp.min(jnp.where(vals == m, lin, N * R))
        vals = jnp.where(lin == sel, NEG, vals)
        rows_acc = jnp.where(opos == k, sel // R, rows_acc)
        rels_acc = jnp.where(opos == k, sel % R, rels_acc)
    rows_ref[...] = rows_acc
    rels_ref[...] = rels_acc


def _rowtopk_kernel(rows_ref, rels_ref, ee_ref, p_ref, pt_ref, pts_ref,
                    vals_ref, js_ref):
    k = pl.program_id(0)
    i = rows_ref[k]
    r = rels_ref[k]
    s = jax.nn.sigmoid(ee_ref[0])
    ridx = jax.lax.broadcasted_iota(jnp.int32, (1, R), 1)
    p_i = jnp.sum(jnp.where(ridx == r, p_ref[0], 0.0))
    cols = jax.lax.broadcasted_iota(jnp.int32, (1, N), 1)
    score = p_i * jnp.where(cols < i, s * pt_ref[0], s * pts_ref[0])
    masked = jnp.where(score > THR, score, NEG)
    opos = jax.lax.broadcasted_iota(jnp.int32, (1, 128), 1)
    vals_acc = jnp.full((1, 128), NEG, jnp.float32)
    js_acc = jnp.zeros((1, 128), jnp.int32)
    for t in range(K):
        m = jnp.max(masked)
        sel = jnp.min(jnp.where(masked == m, cols, N))
        masked = jnp.where(cols == sel, NEG, masked)
        jsel = sel + jnp.where(sel >= i, 1, 0)
        vals_acc = jnp.where(opos == t, m, vals_acc)
        js_acc = jnp.where(opos == t, jsel, js_acc)
    vals_ref[0] = vals_acc
    js_ref[0] = js_acc


def _merge_kernel(vals_ref, js_ref, rows_ref, rels_ref,
                  i_ref, r_ref, j_ref, v_ref):
    vals = vals_ref[...]
    js = js_ref[...]
    rows = rows_ref[...]
    rels = rels_ref[...]
    lin = (
        jax.lax.broadcasted_iota(jnp.int32, (K, 128), 0) * 128
        + jax.lax.broadcasted_iota(jnp.int32, (K, 128), 1)
    )
    opos = (
        jax.lax.broadcasted_iota(jnp.int32, (8, 128), 0) * 128
        + jax.lax.broadcasted_iota(jnp.int32, (8, 128), 1)
    )
    i_acc = jnp.zeros((8, 128), jnp.int32)
    r_acc = jnp.zeros((8, 128), jnp.int32)
    j_acc = jnp.zeros((8, 128), jnp.int32)
    v_acc = jnp.full((8, 128), NEG, jnp.float32)
    for t in range(K):
        m = jnp.max(vals)
        sel = jnp.min(jnp.where(vals == m, lin, K * 128))
        selrow = sel // 128
        vals = jnp.where(lin == sel, NEG, vals)
        jv = jnp.sum(jnp.where(lin == sel, js, 0))
        iv = jnp.sum(jnp.where(opos == selrow, rows, 0))
        rv = jnp.sum(jnp.where(opos == selrow, rels, 0))
        i_acc = jnp.where(opos == t, iv, i_acc)
        r_acc = jnp.where(opos == t, rv, r_acc)
        j_acc = jnp.where(opos == t, jv, j_acc)
        v_acc = jnp.where(opos == t, m, v_acc)
    i_ref[...] = i_acc
    r_ref[...] = r_acc
    j_ref[...] = j_acc
    v_ref[...] = v_acc


@jax.jit
def kernel(ee_scores, er_scores, entity_labels, relation_types):
    p = jax.nn.sigmoid(er_scores)
    pt = p.T
    pts = jnp.concatenate([pt[:, 1:], jnp.zeros((R, 1), jnp.float32)], axis=1)

    rowmax = pl.pallas_call(
        _rowmax_kernel,
        grid=(N // BI,),
        in_specs=[
            pl.BlockSpec((BI, N), lambda b: (b, 0)),
            pl.BlockSpec((BI, R), lambda b: (b, 0)),
            pl.BlockSpec((R, N), lambda b: (0, 0)),
            pl.BlockSpec((R, N), lambda b: (0, 0)),
        ],
        out_specs=pl.BlockSpec((BI, R), lambda b: (b, 0)),
        out_shape=jax.ShapeDtypeStruct((N, R), jnp.float32),
    )(ee_scores, p, pt, pts)

    rows8, rels8 = pl.pallas_call(
        _pairsel_kernel,
        out_shape=(
            jax.ShapeDtypeStruct((8, 128), jnp.int32),
            jax.ShapeDtypeStruct((8, 128), jnp.int32),
        ),
    )(rowmax)
    rows20 = rows8[0, :K]
    rels20 = rels8[0, :K]

    ee3 = ee_scores.reshape(N, 1, N)
    p3 = p.reshape(N, 1, R)
    pt3 = pt.reshape(R, 1, N)
    pts3 = pts.reshape(R, 1, N)
    grid_spec = pltpu.PrefetchScalarGridSpec(
        num_scalar_prefetch=2,
        grid=(K,),
        in_specs=[
            pl.BlockSpec((1, 1, N), lambda k, rows, rels: (rows[k], 0, 0)),
            pl.BlockSpec((1, 1, R), lambda k, rows, rels: (rows[k], 0, 0)),
            pl.BlockSpec((1, 1, N), lambda k, rows, rels: (rels[k], 0, 0)),
            pl.BlockSpec((1, 1, N), lambda k, rows, rels: (rels[k], 0, 0)),
        ],
        out_specs=[
            pl.BlockSpec((1, 1, 128), lambda k, rows, rels: (k, 0, 0)),
            pl.BlockSpec((1, 1, 128), lambda k, rows, rels: (k, 0, 0)),
        ],
    )
    cvals, cjs = pl.pallas_call(
        _rowtopk_kernel,
        grid_spec=grid_spec,
        out_shape=(
            jax.ShapeDtypeStruct((K, 1, 128), jnp.float32),
            jax.ShapeDtypeStruct((K, 1, 128), jnp.int32),
        ),
    )(rows20, rels20, ee3, p3, pt3, pts3)

    i8, r8, j8, v8 = pl.pallas_call(
        _merge_kernel,
        out_shape=(
            jax.ShapeDtypeStruct((8, 128), jnp.int32),
            jax.ShapeDtypeStruct((8, 128), jnp.int32),
            jax.ShapeDtypeStruct((8, 128), jnp.int32),
            jax.ShapeDtypeStruct((8, 128), jnp.float32),
        ),
    )(cvals.reshape(K, 128), cjs.reshape(K, 128), rows8, rels8)

    subjects = jnp.take(entity_labels, i8[0, :K])
    relations = jnp.take(relation_types, r8[0, :K])
    objects = jnp.take(entity_labels, j8[0, :K])
    return subjects, relations, objects, v8[0, :K]


# speedup vs baseline: 152.2825x; 1.4965x over previous
import functools

import jax
import jax.numpy as jnp
from jax.experimental import pallas as pl
from jax.experimental.pallas import tpu as pltpu

N = 5000
R = 16
K = 20
BI = 200
BC = 1024
NCB = -(-N // BC)
NPAD = NCB * BC
THR = 0.5
NEG = float("-inf")


def _rowmax_kernel(ee_ref, p_ref, pt_ref, pts_ref, out_ref):
    rb = pl.program_id(0)
    cb = pl.program_id(1)
    cols = cb * BC + jax.lax.broadcasted_iota(jnp.int32, (BI, BC), 1)
    s = jnp.where(cols < N, jax.nn.sigmoid(ee_ref[...]), 0.0)
    pure_a = (cb + 1) * BC <= rb * BI
    pure_b = cb * BC >= rb * BI + BI - 1

    @pl.when(cb == 0)
    def _():
        out_ref[...] = jnp.full((BI, R), NEG, jnp.float32)

    @pl.when(pure_a)
    def _():
        for r in range(R):
            m = jnp.max(s * pt_ref[r : r + 1, :], axis=1, keepdims=True)
            out_ref[:, r : r + 1] = jnp.maximum(out_ref[:, r : r + 1], m)

    @pl.when(pure_b)
    def _():
        for r in range(R):
            m = jnp.max(s * pts_ref[r : r + 1, :], axis=1, keepdims=True)
            out_ref[:, r : r + 1] = jnp.maximum(out_ref[:, r : r + 1], m)

    @pl.when(jnp.logical_not(jnp.logical_or(pure_a, pure_b)))
    def _():
        rows = rb * BI + jax.lax.broadcasted_iota(jnp.int32, (BI, BC), 0)
        use_a = cols < rows
        for r in range(R):
            y = jnp.where(use_a, s * pt_ref[r : r + 1, :], s * pts_ref[r : r + 1, :])
            m = jnp.max(y, axis=1, keepdims=True)
            out_ref[:, r : r + 1] = jnp.maximum(out_ref[:, r : r + 1], m)

    @pl.when(cb == NCB - 1)
    def _():
        m = out_ref[...] * p_ref[...]
        out_ref[...] = jnp.where(m > THR, m, NEG)


def _pairsel_kernel(rm_ref, rows_ref, rels_ref):
    vals = rm_ref[...]
    lin = (
        jax.lax.broadcasted_iota(jnp.int32, (N, R), 0) * R
        + jax.lax.broadcasted_iota(jnp.int32, (N, R), 1)
    )
    opos = (
        jax.lax.broadcasted_iota(jnp.int32, (8, 128), 0) * 128
        + jax.lax.broadcasted_iota(jnp.int32, (8, 128), 1)
    )
    rows_acc = jnp.zeros((8, 128), jnp.int32)
    rels_acc = jnp.zeros((8, 128), jnp.int32)
    for k in range(K):
        m = jnp.max(vals)
        sel = jnp.min(jnp.where(vals == m, lin, N * R))
        vals = jnp.where(lin == sel, NEG, vals)
        rows_acc = jnp.where(opos == k, sel // R, rows_acc)
        rels_acc = jnp.where(opos == k, sel % R, rels_acc)
    rows_ref[...] = rows_acc
    rels_ref[...] = rels_acc


def _rowtopk_kernel(rows_ref, rels_ref, ee_ref, p_ref, pt_ref, pts_ref,
                    vals_ref, js_ref):
    k = pl.program_id(0)
    i = rows_ref[k]
    r = rels_ref[k]
    s = jax.nn.sigmoid(ee_ref[0])
    ridx = jax.lax.broadcasted_iota(jnp.int32, (1, R), 1)
    p_i = jnp.sum(jnp.where(ridx == r, p_ref[0], 0.0))
    cols = jax.lax.broadcasted_iota(jnp.int32, (1, N), 1)
    score = p_i * jnp.where(cols < i, s * pt_ref[0], s * pts_ref[0])
    masked = jnp.where(score > THR, score, NEG)
    opos = jax.lax.broadcasted_iota(jnp.int32, (1, 128), 1)
    vals_acc = jnp.full((1, 128), NEG, jnp.float32)
    js_acc = jnp.zeros((1, 128), jnp.int32)
    for t in range(K):
        m = jnp.max(masked)
        sel = jnp.min(jnp.where(masked == m, cols, N))
        masked = jnp.where(cols == sel, NEG, masked)
        jsel = sel + jnp.where(sel >= i, 1, 0)
        vals_acc = jnp.where(opos == t, m, vals_acc)
        js_acc = jnp.where(opos == t, jsel, js_acc)
    vals_ref[0] = vals_acc
    js_ref[0] = js_acc


def _merge_kernel(vals_ref, js_ref, rows_ref, rels_ref,
                  i_ref, r_ref, j_ref, v_ref):
    vals = vals_ref[...]
    js = js_ref[...]
    rows = rows_ref[...]
    rels = rels_ref[...]
    lin = (
        jax.lax.broadcasted_iota(jnp.int32, (K, 128), 0) * 128
        + jax.lax.broadcasted_iota(jnp.int32, (K, 128), 1)
    )
    opos = (
        jax.lax.broadcasted_iota(jnp.int32, (8, 128), 0) * 128
        + jax.lax.broadcasted_iota(jnp.int32, (8, 128), 1)
    )
    i_acc = jnp.zeros((8, 128), jnp.int32)
    r_acc = jnp.zeros((8, 128), jnp.int32)
    j_acc = jnp.zeros((8, 128), jnp.int32)
    v_acc = jnp.full((8, 128), NEG, jnp.float32)
    for t in range(K):
        m = jnp.max(vals)
        sel = jnp.min(jnp.where(vals == m, lin, K * 128))
        selrow = sel // 128
        vals = jnp.where(lin == sel, NEG, vals)
        jv = jnp.sum(jnp.where(lin == sel, js, 0))
        iv = jnp.sum(jnp.where(opos == selrow, rows, 0))
        rv = jnp.sum(jnp.where(opos == selrow, rels, 0))
        i_acc = jnp.where(opos == t, iv, i_acc)
        r_acc = jnp.where(opos == t, rv, r_acc)
        j_acc = jnp.where(opos == t, jv, j_acc)
        v_acc = jnp.where(opos == t, m, v_acc)
    i_ref[...] = i_acc
    r_ref[...] = r_acc
    j_ref[...] = j_acc
    v_ref[...] = v_acc


@jax.jit
def kernel(ee_scores, er_scores, entity_labels, relation_types):
    p = jax.nn.sigmoid(er_scores)
    pt = p.T
    pts = jnp.concatenate([pt[:, 1:], jnp.zeros((R, 1), jnp.float32)], axis=1)
    zpad = jnp.zeros((R, NPAD - N), jnp.float32)
    ptp = jnp.concatenate([pt, zpad], axis=1)
    ptsp = jnp.concatenate([pts, zpad], axis=1)

    rowmax = pl.pallas_call(
        _rowmax_kernel,
        grid=(N // BI, NCB),
        in_specs=[
            pl.BlockSpec((BI, BC), lambda b, c: (b, c)),
            pl.BlockSpec((BI, R), lambda b, c: (b, 0)),
            pl.BlockSpec((R, BC), lambda b, c: (0, c)),
            pl.BlockSpec((R, BC), lambda b, c: (0, c)),
        ],
        out_specs=pl.BlockSpec((BI, R), lambda b, c: (b, 0)),
        out_shape=jax.ShapeDtypeStruct((N, R), jnp.float32),
    )(ee_scores, p, ptp, ptsp)

    rows8, rels8 = pl.pallas_call(
        _pairsel_kernel,
        out_shape=(
            jax.ShapeDtypeStruct((8, 128), jnp.int32),
            jax.ShapeDtypeStruct((8, 128), jnp.int32),
        ),
    )(rowmax)
    rows20 = rows8[0, :K]
    rels20 = rels8[0, :K]

    ee3 = ee_scores.reshape(N, 1, N)
    p3 = p.reshape(N, 1, R)
    pt3 = pt.reshape(R, 1, N)
    pts3 = pts.reshape(R, 1, N)
    grid_spec = pltpu.PrefetchScalarGridSpec(
        num_scalar_prefetch=2,
        grid=(K,),
        in_specs=[
            pl.BlockSpec((1, 1, N), lambda k, rows, rels: (rows[k], 0, 0)),
            pl.BlockSpec((1, 1, R), lambda k, rows, rels: (rows[k], 0, 0)),
            pl.BlockSpec((1, 1, N), lambda k, rows, rels: (rels[k], 0, 0)),
            pl.BlockSpec((1, 1, N), lambda k, rows, rels: (rels[k], 0, 0)),
        ],
        out_specs=[
            pl.BlockSpec((1, 1, 128), lambda k, rows, rels: (k, 0, 0)),
            pl.BlockSpec((1, 1, 128), lambda k, rows, rels: (k, 0, 0)),
        ],
    )
    cvals, cjs = pl.pallas_call(
        _rowtopk_kernel,
        grid_spec=grid_spec,
        out_shape=(
            jax.ShapeDtypeStruct((K, 1, 128), jnp.float32),
            jax.ShapeDtypeStruct((K, 1, 128), jnp.int32),
        ),
    )(rows20, rels20, ee3, p3, pt3, pts3)

    i8, r8, j8, v8 = pl.pallas_call(
        _merge_kernel,
        out_shape=(
            jax.ShapeDtypeStruct((8, 128), jnp.int32),
            jax.ShapeDtypeStruct((8, 128), jnp.int32),
            jax.ShapeDtypeStruct((8, 128), jnp.int32),
            jax.ShapeDtypeStruct((8, 128), jnp.float32),
        ),
    )(cvals.reshape(K, 128), cjs.reshape(K, 128), rows8, rels8)

    subjects = jnp.take(entity_labels, i8[0, :K])
    relations = jnp.take(relation_types, r8[0, :K])
    objects = jnp.take(entity_labels, j8[0, :K])
    return subjects, relations, objects, v8[0, :K]


# speedup vs baseline: 221.1577x; 1.4523x over previous
import jax
import jax.numpy as jnp
from jax.experimental import pallas as pl
from jax.experimental.pallas import tpu as pltpu

N = 5000
R = 16
K = 20
BI = 200
BC = 1024
NCB = -(-N // BC)
NPAD = NCB * BC
THR = 0.5
NEG = float("-inf")


def _rowmax_kernel(ee_ref, p_ref, pt_ref, pts_ref, out_ref):
    rb = pl.program_id(0)
    cb = pl.program_id(1)
    cols = cb * BC + jax.lax.broadcasted_iota(jnp.int32, (BI, BC), 1)
    s = jnp.where(cols < N, jax.nn.sigmoid(ee_ref[...]), 0.0)
    pure_a = (cb + 1) * BC <= rb * BI
    pure_b = cb * BC >= rb * BI + BI - 1

    @pl.when(cb == 0)
    def _():
        out_ref[...] = jnp.full((BI, R), NEG, jnp.float32)

    @pl.when(pure_a)
    def _():
        for r in range(R):
            m = jnp.max(s * pt_ref[r : r + 1, :], axis=1, keepdims=True)
            out_ref[:, r : r + 1] = jnp.maximum(out_ref[:, r : r + 1], m)

    @pl.when(pure_b)
    def _():
        for r in range(R):
            m = jnp.max(s * pts_ref[r : r + 1, :], axis=1, keepdims=True)
            out_ref[:, r : r + 1] = jnp.maximum(out_ref[:, r : r + 1], m)

    @pl.when(jnp.logical_not(jnp.logical_or(pure_a, pure_b)))
    def _():
        rows = rb * BI + jax.lax.broadcasted_iota(jnp.int32, (BI, BC), 0)
        use_a = cols < rows
        for r in range(R):
            y = jnp.where(use_a, s * pt_ref[r : r + 1, :], s * pts_ref[r : r + 1, :])
            m = jnp.max(y, axis=1, keepdims=True)
            out_ref[:, r : r + 1] = jnp.maximum(out_ref[:, r : r + 1], m)

    @pl.when(cb == NCB - 1)
    def _():
        m = out_ref[...] * p_ref[...]
        out_ref[...] = jnp.where(m > THR, m, NEG)


def _pairsel_kernel(rmt_ref, rows_ref, rels_ref):
    vals = rmt_ref[...]
    lin = (
        jax.lax.broadcasted_iota(jnp.int32, (R, N), 0) * N
        + jax.lax.broadcasted_iota(jnp.int32, (R, N), 1)
    )
    opos = (
        jax.lax.broadcasted_iota(jnp.int32, (8, 128), 0) * 128
        + jax.lax.broadcasted_iota(jnp.int32, (8, 128), 1)
    )
    rows_acc = jnp.zeros((8, 128), jnp.int32)
    rels_acc = jnp.zeros((8, 128), jnp.int32)
    for k in range(K):
        m = jnp.max(vals)
        sel = jnp.min(jnp.where(vals == m, lin, R * N))
        vals = jnp.where(lin == sel, NEG, vals)
        rows_acc = jnp.where(opos == k, sel % N, rows_acc)
        rels_acc = jnp.where(opos == k, sel // N, rels_acc)
    rows_ref[...] = rows_acc
    rels_ref[...] = rels_acc


def _gather_topk_kernel(rows_ref, rels_ref, *refs):
    ee_rows = refs[:K]
    p_ref, pt_ref, pts_ref = refs[K], refs[K + 1], refs[K + 2]
    i_ref, r_ref, j_ref, v_ref = refs[K + 3 :]
    riota = jax.lax.broadcasted_iota(jnp.int32, (1, R), 1)
    cols1 = jax.lax.broadcasted_iota(jnp.int32, (1, N), 1)
    parts = []
    for k in range(K):
        i_k = rows_ref[k]
        r_k = rels_ref[k]
        s = jax.nn.sigmoid(ee_rows[k][0])
        p_i = jnp.sum(jnp.where(riota == r_k, p_ref[pl.ds(i_k, 1), :], 0.0))
        qa = pt_ref[pl.ds(r_k, 1), :]
        qb = pts_ref[pl.ds(r_k, 1), :]
        sc = p_i * jnp.where(cols1 < i_k, s * qa, s * qb)
        parts.append(jnp.where(sc > THR, sc, NEG))
    vals = jnp.concatenate(parts, axis=0)
    lin = (
        jax.lax.broadcasted_iota(jnp.int32, (K, N), 0) * N
        + jax.lax.broadcasted_iota(jnp.int32, (K, N), 1)
    )
    opos = (
        jax.lax.broadcasted_iota(jnp.int32, (8, 128), 0) * 128
        + jax.lax.broadcasted_iota(jnp.int32, (8, 128), 1)
    )
    i_acc = jnp.zeros((8, 128), jnp.int32)
    r_acc = jnp.zeros((8, 128), jnp.int32)
    j_acc = jnp.zeros((8, 128), jnp.int32)
    v_acc = jnp.full((8, 128), NEG, jnp.float32)
    for t in range(K):
        m = jnp.max(vals)
        sel = jnp.min(jnp.where(vals == m, lin, K * N))
        vals = jnp.where(lin == sel, NEG, vals)
        selk = sel // N
        selc = sel % N
        i_sel = rows_ref[selk]
        r_sel = rels_ref[selk]
        j_sel = selc + jnp.where(selc >= i_sel, 1, 0)
        i_acc = jnp.where(opos == t, i_sel, i_acc)
        r_acc = jnp.where(opos == t, r_sel, r_acc)
        j_acc = jnp.where(opos == t, j_sel, j_acc)
        v_acc = jnp.where(opos == t, m, v_acc)
    i_ref[...] = i_acc
    r_ref[...] = r_acc
    j_ref[...] = j_acc
    v_ref[...] = v_acc


@jax.jit
def kernel(ee_scores, er_scores, entity_labels, relation_types):
    p = jax.nn.sigmoid(er_scores)
    pt = p.T
    pts = jnp.concatenate([pt[:, 1:], jnp.zeros((R, 1), jnp.float32)], axis=1)
    zpad = jnp.zeros((R, NPAD - N), jnp.float32)
    ptp = jnp.concatenate([pt, zpad], axis=1)
    ptsp = jnp.concatenate([pts, zpad], axis=1)

    rowmax = pl.pallas_call(
        _rowmax_kernel,
        grid=(N // BI, NCB),
        in_specs=[
            pl.BlockSpec((BI, BC), lambda b, c: (b, c)),
            pl.BlockSpec((BI, R), lambda b, c: (b, 0)),
            pl.BlockSpec((R, BC), lambda b, c: (0, c)),
            pl.BlockSpec((R, BC), lambda b, c: (0, c)),
        ],
        out_specs=pl.BlockSpec((BI, R), lambda b, c: (b, 0)),
        out_shape=jax.ShapeDtypeStruct((N, R), jnp.float32),
    )(ee_scores, p, ptp, ptsp)

    rows8, rels8 = pl.pallas_call(
        _pairsel_kernel,
        out_shape=(
            jax.ShapeDtypeStruct((8, 128), jnp.int32),
            jax.ShapeDtypeStruct((8, 128), jnp.int32),
        ),
    )(rowmax.T)
    rows20 = rows8[0, :K]
    rels20 = rels8[0, :K]

    ee3 = ee_scores.reshape(N, 1, N)
    ee_specs = [
        pl.BlockSpec((1, 1, N), lambda g, rows, rels, k=k: (rows[k], 0, 0))
        for k in range(K)
    ]
    grid_spec = pltpu.PrefetchScalarGridSpec(
        num_scalar_prefetch=2,
        grid=(1,),
        in_specs=ee_specs
        + [
            pl.BlockSpec((N, R), lambda g, rows, rels: (0, 0)),
            pl.BlockSpec((R, N), lambda g, rows, rels: (0, 0)),
            pl.BlockSpec((R, N), lambda g, rows, rels: (0, 0)),
        ],
        out_specs=[
            pl.BlockSpec((8, 128), lambda g, rows, rels: (0, 0)),
            pl.BlockSpec((8, 128), lambda g, rows, rels: (0, 0)),
            pl.BlockSpec((8, 128), lambda g, rows, rels: (0, 0)),
            pl.BlockSpec((8, 128), lambda g, rows, rels: (0, 0)),
        ],
    )
    i8, r8, j8, v8 = pl.pallas_call(
        _gather_topk_kernel,
        grid_spec=grid_spec,
        out_shape=(
            jax.ShapeDtypeStruct((8, 128), jnp.int32),
            jax.ShapeDtypeStruct((8, 128), jnp.int32),
            jax.ShapeDtypeStruct((8, 128), jnp.int32),
            jax.ShapeDtypeStruct((8, 128), jnp.float32),
        ),
    )(rows20, rels20, *([ee3] * K), p, pt, pts)

    subjects = jnp.take(entity_labels, i8[0, :K])
    relations = jnp.take(relation_types, r8[0, :K])
    objects = jnp.take(entity_labels, j8[0, :K])
    return subjects, relations, objects, v8[0, :K]


# speedup vs baseline: 221.2529x; 1.0004x over previous
import jax
import jax.numpy as jnp
from jax.experimental import pallas as pl
from jax.experimental.pallas import tpu as pltpu

N = 5000
R = 16
K = 20
BI = 200
BC = 1024
NCB = -(-N // BC)
NPAD = NCB * BC
THR = 0.5
NEG = float("-inf")


def _rowmax_kernel(ee_ref, p_ref, pt_ref, pts_ref, out_ref):
    rb = pl.program_id(0)
    cb = pl.program_id(1)
    cols = cb * BC + jax.lax.broadcasted_iota(jnp.int32, (BI, BC), 1)
    s = jnp.where(cols < N, jax.nn.sigmoid(ee_ref[...]), 0.0)
    pure_a = (cb + 1) * BC <= rb * BI
    pure_b = cb * BC >= rb * BI + BI - 1

    @pl.when(cb == 0)
    def _():
        out_ref[...] = jnp.full((BI, R), NEG, jnp.float32)

    @pl.when(pure_a)
    def _():
        for r in range(R):
            m = jnp.max(s * pt_ref[r : r + 1, :], axis=1, keepdims=True)
            out_ref[:, r : r + 1] = jnp.maximum(out_ref[:, r : r + 1], m)

    @pl.when(pure_b)
    def _():
        for r in range(R):
            m = jnp.max(s * pts_ref[r : r + 1, :], axis=1, keepdims=True)
            out_ref[:, r : r + 1] = jnp.maximum(out_ref[:, r : r + 1], m)

    @pl.when(jnp.logical_not(jnp.logical_or(pure_a, pure_b)))
    def _():
        rows = rb * BI + jax.lax.broadcasted_iota(jnp.int32, (BI, BC), 0)
        use_a = cols < rows
        for r in range(R):
            y = jnp.where(use_a, s * pt_ref[r : r + 1, :], s * pts_ref[r : r + 1, :])
            m = jnp.max(y, axis=1, keepdims=True)
            out_ref[:, r : r + 1] = jnp.maximum(out_ref[:, r : r + 1], m)

    @pl.when(cb == NCB - 1)
    def _():
        m = out_ref[...] * p_ref[...]
        out_ref[...] = jnp.where(m > THR, m, NEG)


def _pairsel_kernel(rmt_ref, rows_ref, rels_ref):
    vals = rmt_ref[...]
    lin = (
        jax.lax.broadcasted_iota(jnp.int32, (R, N), 0) * N
        + jax.lax.broadcasted_iota(jnp.int32, (R, N), 1)
    )
    opos = (
        jax.lax.broadcasted_iota(jnp.int32, (8, 128), 0) * 128
        + jax.lax.broadcasted_iota(jnp.int32, (8, 128), 1)
    )
    rows_acc = jnp.zeros((8, 128), jnp.int32)
    rels_acc = jnp.zeros((8, 128), jnp.int32)
    for k in range(K):
        m = jnp.max(vals)
        sel = jnp.min(jnp.where(vals == m, lin, R * N))
        vals = jnp.where(lin == sel, NEG, vals)
        rows_acc = jnp.where(opos == k, sel % N, rows_acc)
        rels_acc = jnp.where(opos == k, sel // N, rels_acc)
    rows_ref[...] = rows_acc
    rels_ref[...] = rels_acc


def _gather_topk_kernel(rows_ref, rels_ref, *refs):
    ee_rows = refs[:K]
    p_ref, pt_ref, pts_ref = refs[K], refs[K + 1], refs[K + 2]
    i_ref, r_ref, j_ref, v_ref = refs[K + 3 :]
    riota = jax.lax.broadcasted_iota(jnp.int32, (1, R), 1)
    cols1 = jax.lax.broadcasted_iota(jnp.int32, (1, N), 1)
    parts = []
    for k in range(K):
        i_k = rows_ref[k]
        r_k = rels_ref[k]
        s = jax.nn.sigmoid(ee_rows[k][0])
        p_i = jnp.sum(jnp.where(riota == r_k, p_ref[pl.ds(i_k, 1), :], 0.0))
        qa = pt_ref[pl.ds(r_k, 1), :]
        qb = pts_ref[pl.ds(r_k, 1), :]
        sc = p_i * jnp.where(cols1 < i_k, s * qa, s * qb)
        parts.append(jnp.where(sc > THR, sc, NEG))
    vals = jnp.concatenate(parts, axis=0)
    lin = (
        jax.lax.broadcasted_iota(jnp.int32, (K, N), 0) * N
        + jax.lax.broadcasted_iota(jnp.int32, (K, N), 1)
    )
    opos = (
        jax.lax.broadcasted_iota(jnp.int32, (8, 128), 0) * 128
        + jax.lax.broadcasted_iota(jnp.int32, (8, 128), 1)
    )
    i_acc = jnp.zeros((8, 128), jnp.int32)
    r_acc = jnp.zeros((8, 128), jnp.int32)
    j_acc = jnp.zeros((8, 128), jnp.int32)
    v_acc = jnp.full((8, 128), NEG, jnp.float32)
    for t in range(K):
        m = jnp.max(vals)
        sel = jnp.min(jnp.where(vals == m, lin, K * N))
        vals = jnp.where(lin == sel, NEG, vals)
        selk = sel // N
        selc = sel % N
        i_sel = rows_ref[selk]
        r_sel = rels_ref[selk]
        j_sel = selc + jnp.where(selc >= i_sel, 1, 0)
        i_acc = jnp.where(opos == t, i_sel, i_acc)
        r_acc = jnp.where(opos == t, r_sel, r_acc)
        j_acc = jnp.where(opos == t, j_sel, j_acc)
        v_acc = jnp.where(opos == t, m, v_acc)
    i_ref[...] = i_acc
    r_ref[...] = r_acc
    j_ref[...] = j_acc
    v_ref[...] = v_acc


@jax.jit
def kernel(ee_scores, er_scores, entity_labels, relation_types):
    p = jax.nn.sigmoid(er_scores)
    pt = p.T
    pts = jnp.concatenate([pt[:, 1:], jnp.zeros((R, 1), jnp.float32)], axis=1)
    zpad = jnp.zeros((R, NPAD - N), jnp.float32)
    ptp = jnp.concatenate([pt, zpad], axis=1)
    ptsp = jnp.concatenate([pts, zpad], axis=1)

    rowmax = pl.pallas_call(
        _rowmax_kernel,
        grid=(N // BI, NCB),
        in_specs=[
            pl.BlockSpec((BI, BC), lambda b, c: (b, c)),
            pl.BlockSpec((BI, R), lambda b, c: (b, 0)),
            pl.BlockSpec((R, BC), lambda b, c: (0, c)),
            pl.BlockSpec((R, BC), lambda b, c: (0, c)),
        ],
        out_specs=pl.BlockSpec((BI, R), lambda b, c: (b, 0)),
        out_shape=jax.ShapeDtypeStruct((N, R), jnp.float32),
        compiler_params=pltpu.CompilerParams(
            dimension_semantics=("parallel", "arbitrary")
        ),
    )(ee_scores, p, ptp, ptsp)

    rows8, rels8 = pl.pallas_call(
        _pairsel_kernel,
        out_shape=(
            jax.ShapeDtypeStruct((8, 128), jnp.int32),
            jax.ShapeDtypeStruct((8, 128), jnp.int32),
        ),
    )(rowmax.T)
    rows20 = rows8[0, :K]
    rels20 = rels8[0, :K]

    ee3 = ee_scores.reshape(N, 1, N)
    ee_specs = [
        pl.BlockSpec((1, 1, N), lambda g, rows, rels, k=k: (rows[k], 0, 0))
        for k in range(K)
    ]
    grid_spec = pltpu.PrefetchScalarGridSpec(
        num_scalar_prefetch=2,
        grid=(1,),
        in_specs=ee_specs
        + [
            pl.BlockSpec((N, R), lambda g, rows, rels: (0, 0)),
            pl.BlockSpec((R, N), lambda g, rows, rels: (0, 0)),
            pl.BlockSpec((R, N), lambda g, rows, rels: (0, 0)),
        ],
        out_specs=[
            pl.BlockSpec((8, 128), lambda g, rows, rels: (0, 0)),
            pl.BlockSpec((8, 128), lambda g, rows, rels: (0, 0)),
            pl.BlockSpec((8, 128), lambda g, rows, rels: (0, 0)),
            pl.BlockSpec((8, 128), lambda g, rows, rels: (0, 0)),
        ],
    )
    i8, r8, j8, v8 = pl.pallas_call(
        _gather_topk_kernel,
        grid_spec=grid_spec,
        out_shape=(
            jax.ShapeDtypeStruct((8, 128), jnp.int32),
            jax.ShapeDtypeStruct((8, 128), jnp.int32),
            jax.ShapeDtypeStruct((8, 128), jnp.int32),
            jax.ShapeDtypeStruct((8, 128), jnp.float32),
        ),
    )(rows20, rels20, *([ee3] * K), p, pt, pts)

    subjects = jnp.take(entity_labels, i8[0, :K])
    relations = jnp.take(relation_types, r8[0, :K])
    objects = jnp.take(entity_labels, j8[0, :K])
    return subjects, relations, objects, v8[0, :K]
